# baseline (speedup 1.0000x reference)
import functools

import jax
import jax.numpy as jnp
from jax import lax
from jax.experimental import pallas as pl
from jax.experimental.pallas import tpu as pltpu
from jax.experimental.pallas import tpu_sc as plsc

N_NODES = 10000
H = 128
O = 128
N_RELS = 8
N_BASES = 4
N_EDGES = 320000

NC = 2
NS = 16
NW = NC * NS
E_PER_W = N_EDGES // NW
CHUNK = 80
N_CHUNKS = E_PER_W // CHUNK
ACC_ROWS = 10240
ROWS_PER_S = ACC_ROWS // NS


def _make_y(x, V, comp):
    nb = 10
    bm = N_NODES // nb

    def body(comp_ref, v_ref, x_ref, y_ref):
        xb = x_ref[...]
        for r in range(N_RELS):
            w = jnp.sum(comp_ref[r][:, None, None] * v_ref[...], axis=0)
            y_ref[r] = jnp.dot(xb, w, preferred_element_type=jnp.float32)

    return pl.pallas_call(
        body,
        grid=(nb,),
        in_specs=[
            pl.BlockSpec((N_RELS, N_BASES), lambda n: (0, 0)),
            pl.BlockSpec((N_BASES, H, O), lambda n: (0, 0, 0)),
            pl.BlockSpec((bm, H), lambda n: (n, 0)),
        ],
        out_specs=pl.BlockSpec((N_RELS, bm, O), lambda n: (0, n, 0)),
        out_shape=jax.ShapeDtypeStruct((N_RELS, N_NODES, O), jnp.float32),
    )(comp, V, x)


def _sc_aggregate(y, src_r, dst_r, et_r):
    mesh = plsc.VectorSubcoreMesh(core_axis_name="c", subcore_axis_name="s",
                                  num_cores=NC, num_subcores=NS)

    @functools.partial(
        pl.kernel,
        mesh=mesh,
        compiler_params=pltpu.CompilerParams(use_tc_tiling_on_sc=False, needs_layout_passes=False),
        out_type=(
            jax.ShapeDtypeStruct((NC, ACC_ROWS, O), jnp.float32),
            jax.ShapeDtypeStruct((NW, ROWS_PER_S, 16), jnp.float32),
        ),
        scratch_types=[
            pltpu.VMEM((E_PER_W,), jnp.int32),
            pltpu.VMEM((2000,), jnp.int32),
            pltpu.VMEM((2, CHUNK), jnp.int32),
            pltpu.VMEM((2, CHUNK, O), jnp.float32),
            pltpu.VMEM((ROWS_PER_S, 16), jnp.float32),
            pltpu.VMEM_SHARED((ACC_ROWS, O), jnp.float32),
            pltpu.SemaphoreType.DMA,
            pltpu.SemaphoreType.DMA,
            pltpu.SemaphoreType.DMA,
            pltpu.SemaphoreType.DMA,
        ],
    )
    def body(y_hbm, src_hbm, dst_hbm, et_hbm, acc_out, deg_out,
             gidx_b, et_b, dst_b, rows_b, hist, acc_sh, gsem, dsem,
             ssem0, ssem1):
        c = lax.axis_index("c")
        s = lax.axis_index("s")
        wid = c * NS + s
        base = s * ROWS_PER_S

        zeros16 = jnp.zeros((16,), jnp.float32)
        ones16 = jnp.ones((16,), jnp.float32)

        def z_rows(i, _):
            rows_b[0, i // 8, pl.ds((i % 8) * 16, 16)] = zeros16
            return 0
        lax.fori_loop(0, CHUNK * (O // 16), z_rows, 0)

        def z_hist(i, _):
            hist[i] = zeros16
            return 0
        lax.fori_loop(0, ROWS_PER_S, z_hist, 0)

        for k in range(ROWS_PER_S // CHUNK):
            pltpu.sync_copy(rows_b.at[0], acc_sh.at[pl.ds(base + k * CHUNK, CHUNK)])

        pltpu.sync_copy(src_hbm.at[wid], gidx_b)
        for blk in range(E_PER_W // 2000):
            pltpu.sync_copy(et_hbm.at[wid, pl.ds(blk * 2000, 2000)], et_b)

            def gidx(i, _, blk=blk):
                sl = pl.ds(blk * 2000 + i * 16, 16)
                gidx_b[sl] = et_b[pl.ds(i * 16, 16)] * N_NODES + gidx_b[sl]
                return 0
            lax.fori_loop(0, 125, gidx, 0)

        plsc.subcore_barrier()

        pltpu.async_copy(dst_hbm.at[wid, 0], dst_b.at[0], dsem)
        pltpu.async_copy(y_hbm.at[gidx_b.at[pl.ds(0, CHUNK)]], rows_b.at[0],
                         gsem)

        ssems = (ssem0, ssem1)

        def outer(t, _):
            for b in range(2):
                j = 2 * t + b
                o = 1 - b
                pltpu.make_async_copy(dst_hbm.at[wid, 0], dst_b.at[b],
                                      dsem).wait()
                pltpu.make_async_copy(
                    y_hbm.at[gidx_b.at[pl.ds(0, CHUNK)]], rows_b.at[b],
                    gsem).wait()

                pltpu.async_copy(rows_b.at[b], acc_sh.at[dst_b.at[b]],
                                 ssems[b], add=True)

                @pl.when(j > 0)
                def _():
                    pltpu.make_async_copy(
                        rows_b.at[o], acc_sh.at[dst_b.at[o]],
                        ssems[o]).wait()

                @pl.when(j < N_CHUNKS - 1)
                def _():
                    pltpu.async_copy(dst_hbm.at[wid, j + 1], dst_b.at[o],
                                     dsem)
                    idx = gidx_b.at[pl.ds((j + 1) * CHUNK, CHUNK)]
                    pltpu.async_copy(y_hbm.at[idx], rows_b.at[o], gsem)

                def hist_up(i, _):
                    d = dst_b[b, pl.ds(i * 16, 16)]
                    plsc.addupdate_scatter(hist, [d >> 4, d & 15], ones16)
                    return 0
                lax.fori_loop(0, CHUNK // 16, hist_up, 0)
            return 0
        lax.fori_loop(0, N_CHUNKS // 2, outer, 0)

        last = N_CHUNKS - 1
        lb = last % 2
        lo = 1 - lb
        pltpu.make_async_copy(dst_hbm.at[wid, 0], dst_b.at[lb], dsem).wait()
        pltpu.make_async_copy(
            y_hbm.at[gidx_b.at[pl.ds(0, CHUNK)]], rows_b.at[lb], gsem).wait()
        pltpu.async_copy(rows_b.at[lb], acc_sh.at[dst_b.at[lb]], ssems[lb],
                         add=True)
        pltpu.make_async_copy(rows_b.at[lo], acc_sh.at[dst_b.at[lo]],
                              ssems[lo]).wait()

        def hist_tail(i, _):
            d = dst_b[lb, pl.ds(i * 16, 16)]
            plsc.addupdate_scatter(hist, [d >> 4, d & 15], ones16)
            return 0
        lax.fori_loop(0, CHUNK // 16, hist_tail, 0)

        pltpu.make_async_copy(rows_b.at[lb], acc_sh.at[dst_b.at[lb]],
                              ssems[lb]).wait()

        plsc.subcore_barrier()

        pltpu.sync_copy(acc_sh.at[pl.ds(base, ROWS_PER_S)],
                        acc_out.at[c, pl.ds(base, ROWS_PER_S)])
        pltpu.sync_copy(hist, deg_out.at[wid])

    return body(y, src_r, dst_r, et_r)


def _finalize(acc, deg):
    nb = 5
    bm = ACC_ROWS // nb

    def body(a_ref, d_ref, o_ref):
        d = jnp.sum(d_ref[...], axis=0)[:, None]
        norm = 1.0 / jnp.maximum(d, 1.0)
        o_ref[...] = jnp.maximum((a_ref[0] + a_ref[1]) * norm, 0.0)

    return pl.pallas_call(
        body,
        grid=(nb,),
        in_specs=[
            pl.BlockSpec((NC, bm, O), lambda n: (0, n, 0)),
            pl.BlockSpec((NW, bm), lambda n: (0, n)),
        ],
        out_specs=pl.BlockSpec((bm, O), lambda n: (n, 0)),
        out_shape=jax.ShapeDtypeStruct((ACC_ROWS, O), jnp.float32),
    )(acc, deg)


def kernel(x, edge_index, edge_type, V, comp):
    src_r = edge_index[0].astype(jnp.int32).reshape(NW, E_PER_W)
    dst_r = edge_index[1].astype(jnp.int32).reshape(NW, N_CHUNKS, CHUNK)
    et_r = edge_type.astype(jnp.int32).reshape(NW, E_PER_W)
    y = _make_y(x, V, comp).reshape(N_RELS * N_NODES, O)
    acc, deg = _sc_aggregate(y, src_r, dst_r, et_r)
    return _finalize(acc, deg.reshape(NW, ACC_ROWS))[:N_NODES]


# speedup vs baseline: 1.3257x; 1.3257x over previous
import functools

import jax
import jax.numpy as jnp
from jax import lax
from jax.experimental import pallas as pl
from jax.experimental.pallas import tpu as pltpu
from jax.experimental.pallas import tpu_sc as plsc

N_NODES = 10000
H = 128
O = 128
N_RELS = 8
N_BASES = 4
N_EDGES = 320000

NC = 2
NS = 16
NW = NC * NS
E_PER_W = N_EDGES // NW
CHUNK = 40
N_CHUNKS = E_PER_W // CHUNK
ACC_ROWS = 10240
ROWS_PER_S = ACC_ROWS // NS


def _make_y(x, V, comp):
    nb = 10
    bm = N_NODES // nb

    def body(comp_ref, v_ref, x_ref, y_ref):
        xb = x_ref[...]
        for r in range(N_RELS):
            w = jnp.sum(comp_ref[r][:, None, None] * v_ref[...], axis=0)
            y_ref[r] = jnp.dot(xb, w, preferred_element_type=jnp.float32)

    return pl.pallas_call(
        body,
        grid=(nb,),
        in_specs=[
            pl.BlockSpec((N_RELS, N_BASES), lambda n: (0, 0)),
            pl.BlockSpec((N_BASES, H, O), lambda n: (0, 0, 0)),
            pl.BlockSpec((bm, H), lambda n: (n, 0)),
        ],
        out_specs=pl.BlockSpec((N_RELS, bm, O), lambda n: (0, n, 0)),
        out_shape=jax.ShapeDtypeStruct((N_RELS, N_NODES, O), jnp.float32),
    )(comp, V, x)


def _sc_aggregate(y, gidx_r, dst_r):
    mesh = plsc.VectorSubcoreMesh(core_axis_name="c", subcore_axis_name="s",
                                  num_cores=NC, num_subcores=NS)

    RING = 4

    @functools.partial(
        pl.kernel,
        mesh=mesh,
        compiler_params=pltpu.CompilerParams(use_tc_tiling_on_sc=False,
                                             needs_layout_passes=False),
        out_type=(
            jax.ShapeDtypeStruct((NC, ACC_ROWS, O), jnp.float32),
            jax.ShapeDtypeStruct((NW, ROWS_PER_S, 16), jnp.float32),
        ),
        scratch_types=[
            pltpu.VMEM((E_PER_W,), jnp.int32),
            pltpu.VMEM((RING * CHUNK,), jnp.int32),
            pltpu.VMEM((RING, CHUNK, O), jnp.float32),
            pltpu.VMEM((ROWS_PER_S, 16), jnp.float32),
            pltpu.VMEM_SHARED((ACC_ROWS, O), jnp.float32),
            pltpu.SemaphoreType.DMA,
            pltpu.SemaphoreType.DMA,
        ],
    )
    def body(y_hbm, gidx_hbm, dst_hbm, acc_out, deg_out,
             gidx_b, dst_b, rows_b, hist, acc_sh, gsem, dsem):
        c = lax.axis_index("c")
        s = lax.axis_index("s")
        wid = c * NS + s
        base = s * ROWS_PER_S

        zeros16 = jnp.zeros((16,), jnp.float32)
        ones16 = jnp.ones((16,), jnp.float32)

        def z_rows(i, _):
            rows_b[0, i // 8, pl.ds((i % 8) * 16, 16)] = zeros16
            return 0
        lax.fori_loop(0, CHUNK * (O // 16), z_rows, 0)

        def z_hist(i, _):
            hist[i] = zeros16
            return 0
        lax.fori_loop(0, ROWS_PER_S, z_hist, 0)

        for k in range(ROWS_PER_S // CHUNK):
            pltpu.sync_copy(rows_b.at[0],
                            acc_sh.at[pl.ds(base + k * CHUNK, CHUNK)])

        pltpu.sync_copy(gidx_hbm.at[wid], gidx_b)

        plsc.subcore_barrier()

        for p in range(RING - 1):
            pltpu.async_copy(dst_hbm.at[wid, p],
                             dst_b.at[pl.ds(p * CHUNK, CHUNK)], dsem)
            pltpu.async_copy(y_hbm.at[gidx_b.at[pl.ds(p * CHUNK, CHUNK)]],
                             rows_b.at[p], gsem)

        def hist_pair(b):
            def hist_up(i, _):
                d = dst_b[pl.ds((b - 1) * CHUNK + i * 16, 16)]
                plsc.addupdate_scatter(hist, [d >> 4, d & 15], ones16)
                return 0
            lax.fori_loop(0, 2 * CHUNK // 16, hist_up, 0)

        def do_chunk(j, b):
            pltpu.make_async_copy(dst_hbm.at[wid, 0],
                                  dst_b.at[pl.ds(b * CHUNK, CHUNK)],
                                  dsem).wait()
            pltpu.make_async_copy(
                y_hbm.at[gidx_b.at[pl.ds(0, CHUNK)]], rows_b.at[b],
                gsem).wait()

            if b % 2 == 1:
                hist_pair(b)

            @pl.when(j + RING - 1 < N_CHUNKS)
            def _():
                jj = j + RING - 1
                nb_ = (b + RING - 1) % RING
                pltpu.async_copy(dst_hbm.at[wid, jj],
                                 dst_b.at[pl.ds(nb_ * CHUNK, CHUNK)], dsem)
                idx = gidx_b.at[pl.ds(jj * CHUNK, CHUNK)]
                pltpu.async_copy(y_hbm.at[idx], rows_b.at[nb_], gsem)

            pltpu.sync_copy(rows_b.at[b],
                            acc_sh.at[dst_b.at[pl.ds(b * CHUNK, CHUNK)]],
                            add=True)

        def outer(t, _):
            for u in range(RING):
                do_chunk(RING * t + u, u)
            return 0
        lax.fori_loop(0, N_CHUNKS // RING, outer, 0)

        for j in range((N_CHUNKS // RING) * RING, N_CHUNKS):
            do_chunk(j, j % RING)

        plsc.subcore_barrier()

        pltpu.sync_copy(acc_sh.at[pl.ds(base, ROWS_PER_S)],
                        acc_out.at[c, pl.ds(base, ROWS_PER_S)])
        pltpu.sync_copy(hist, deg_out.at[wid])

    return body(y, gidx_r, dst_r)


def _finalize(acc, deg):
    nb = 5
    bm = ACC_ROWS // nb

    def body(a_ref, d_ref, o_ref):
        d = jnp.sum(d_ref[...], axis=0)[:, None]
        norm = 1.0 / jnp.maximum(d, 1.0)
        o_ref[...] = jnp.maximum((a_ref[0] + a_ref[1]) * norm, 0.0)

    return pl.pallas_call(
        body,
        grid=(nb,),
        in_specs=[
            pl.BlockSpec((NC, bm, O), lambda n: (0, n, 0)),
            pl.BlockSpec((NW, bm), lambda n: (0, n)),
        ],
        out_specs=pl.BlockSpec((bm, O), lambda n: (n, 0)),
        out_shape=jax.ShapeDtypeStruct((ACC_ROWS, O), jnp.float32),
    )(acc, deg)


def kernel(x, edge_index, edge_type, V, comp):
    gidx_r = (edge_type.astype(jnp.int32) * N_NODES
              + edge_index[0].astype(jnp.int32)).reshape(NW, E_PER_W)
    dst_r = edge_index[1].astype(jnp.int32).reshape(NW, N_CHUNKS, CHUNK)
    y = _make_y(x, V, comp).reshape(N_RELS * N_NODES, O)
    acc, deg = _sc_aggregate(y, gidx_r, dst_r)
    return _finalize(acc, deg.reshape(NW, ACC_ROWS))[:N_NODES]


# speedup vs baseline: 1.5117x; 1.1403x over previous
import functools

import jax
import jax.numpy as jnp
from jax import lax
from jax.experimental import pallas as pl
from jax.experimental.pallas import tpu as pltpu
from jax.experimental.pallas import tpu_sc as plsc

N_NODES = 10000
H = 128
O = 128
N_RELS = 8
N_BASES = 4
N_EDGES = 320000

NC = 2
NS = 16
NW = NC * NS
E_PER_W = N_EDGES // NW
CHUNK = 40
N_CHUNKS = E_PER_W // CHUNK
ACC_ROWS = 10240
ROWS_PER_S = ACC_ROWS // NS


def _make_y(x, V, comp):
    nb = 10
    bm = N_NODES // nb

    def body(comp_ref, v_ref, x_ref, y_ref):
        xb = x_ref[...]
        for r in range(N_RELS):
            w = jnp.sum(comp_ref[r][:, None, None] * v_ref[...], axis=0)
            y_ref[r] = jnp.dot(xb, w, preferred_element_type=jnp.float32)

    return pl.pallas_call(
        body,
        grid=(nb,),
        in_specs=[
            pl.BlockSpec((N_RELS, N_BASES), lambda n: (0, 0)),
            pl.BlockSpec((N_BASES, H, O), lambda n: (0, 0, 0)),
            pl.BlockSpec((bm, H), lambda n: (n, 0)),
        ],
        out_specs=pl.BlockSpec((N_RELS, bm, O), lambda n: (0, n, 0)),
        out_shape=jax.ShapeDtypeStruct((N_RELS, N_NODES, O), jnp.float32),
    )(comp, V, x)


def _sc_aggregate(y, gidx_r, dst_r):
    mesh = plsc.VectorSubcoreMesh(core_axis_name="c", subcore_axis_name="s",
                                  num_cores=NC, num_subcores=NS)

    RING = 5

    @functools.partial(
        pl.kernel,
        mesh=mesh,
        compiler_params=pltpu.CompilerParams(use_tc_tiling_on_sc=False,
                                             needs_layout_passes=False),
        out_type=(
            jax.ShapeDtypeStruct((NC, ACC_ROWS, O), jnp.float32),
            jax.ShapeDtypeStruct((NW, ACC_ROWS), jnp.float32),
        ),
        scratch_types=[
            pltpu.VMEM((E_PER_W,), jnp.int32),
            pltpu.VMEM((RING * CHUNK + 8,), jnp.int32),
            pltpu.VMEM((RING, CHUNK, O), jnp.float32),
            pltpu.VMEM((ACC_ROWS,), jnp.float32),
            pltpu.VMEM_SHARED((ACC_ROWS, O), jnp.float32),
            pltpu.SemaphoreType.DMA,
            pltpu.SemaphoreType.DMA,
        ],
    )
    def body(y_hbm, gidx_hbm, dst_hbm, acc_out, deg_out,
             gidx_b, dst_b, rows_b, hist, acc_sh, gsem, dsem):
        c = lax.axis_index("c")
        s = lax.axis_index("s")
        wid = c * NS + s
        base = s * ROWS_PER_S

        zeros16 = jnp.zeros((16,), jnp.float32)
        ones16 = jnp.ones((16,), jnp.float32)

        def z_rows(i, _):
            rows_b[0, i // 8, pl.ds((i % 8) * 16, 16)] = zeros16
            return 0
        lax.fori_loop(0, CHUNK * (O // 16), z_rows, 0)

        def z_hist(i, _):
            hist[pl.ds(i * 16, 16)] = zeros16
            return 0
        lax.fori_loop(0, ACC_ROWS // 16, z_hist, 0)

        for k in range(ROWS_PER_S // CHUNK):
            pltpu.sync_copy(rows_b.at[0],
                            acc_sh.at[pl.ds(base + k * CHUNK, CHUNK)])

        pltpu.sync_copy(gidx_hbm.at[wid], gidx_b)

        plsc.subcore_barrier()

        for p in range(RING - 1):
            pltpu.async_copy(dst_hbm.at[wid, p],
                             dst_b.at[pl.ds(p * CHUNK, CHUNK)], dsem)
            pltpu.async_copy(y_hbm.at[gidx_b.at[pl.ds(p * CHUNK, CHUNK)]],
                             rows_b.at[p], gsem)

        mask8 = lax.iota(jnp.int32, 16) < 8

        def hist_chunk(b):
            for i in range(2):
                d = dst_b[pl.ds(b * CHUNK + i * 16, 16)]
                plsc.addupdate_scatter(hist, [d], ones16)
            d = dst_b[pl.ds(b * CHUNK + 32, 16)]
            plsc.addupdate_scatter(hist, [d], ones16, mask=mask8)

        def do_chunk(j, b):
            pltpu.make_async_copy(dst_hbm.at[wid, 0],
                                  dst_b.at[pl.ds(b * CHUNK, CHUNK)],
                                  dsem).wait()
            pltpu.make_async_copy(
                y_hbm.at[gidx_b.at[pl.ds(0, CHUNK)]], rows_b.at[b],
                gsem).wait()

            hist_chunk(b)

            @pl.when(j + RING - 1 < N_CHUNKS)
            def _():
                jj = j + RING - 1
                nb_ = (b + RING - 1) % RING
                pltpu.async_copy(dst_hbm.at[wid, jj],
                                 dst_b.at[pl.ds(nb_ * CHUNK, CHUNK)], dsem)
                idx = gidx_b.at[pl.ds(jj * CHUNK, CHUNK)]
                pltpu.async_copy(y_hbm.at[idx], rows_b.at[nb_], gsem)

            pltpu.sync_copy(rows_b.at[b],
                            acc_sh.at[dst_b.at[pl.ds(b * CHUNK, CHUNK)]],
                            add=True)

        def outer(t, _):
            for u in range(RING):
                do_chunk(RING * t + u, u)
            return 0
        lax.fori_loop(0, N_CHUNKS // RING, outer, 0)

        for j in range((N_CHUNKS // RING) * RING, N_CHUNKS):
            do_chunk(j, j % RING)

        plsc.subcore_barrier()

        pltpu.sync_copy(acc_sh.at[pl.ds(base, ROWS_PER_S)],
                        acc_out.at[c, pl.ds(base, ROWS_PER_S)])
        pltpu.sync_copy(hist, deg_out.at[wid])

    return body(y, gidx_r, dst_r)


def _finalize(acc, deg):
    nb = 5
    bm = ACC_ROWS // nb

    def body(a_ref, d_ref, o_ref):
        d = jnp.sum(d_ref[...], axis=0)[:, None]
        norm = 1.0 / jnp.maximum(d, 1.0)
        o_ref[...] = jnp.maximum((a_ref[0] + a_ref[1]) * norm, 0.0)

    return pl.pallas_call(
        body,
        grid=(nb,),
        in_specs=[
            pl.BlockSpec((NC, bm, O), lambda n: (0, n, 0)),
            pl.BlockSpec((NW, bm), lambda n: (0, n)),
        ],
        out_specs=pl.BlockSpec((bm, O), lambda n: (n, 0)),
        out_shape=jax.ShapeDtypeStruct((ACC_ROWS, O), jnp.float32),
    )(acc, deg)


def kernel(x, edge_index, edge_type, V, comp):
    gidx_r = (edge_type.astype(jnp.int32) * N_NODES
              + edge_index[0].astype(jnp.int32)).reshape(NW, E_PER_W)
    dst_r = edge_index[1].astype(jnp.int32).reshape(NW, N_CHUNKS, CHUNK)
    y = _make_y(x, V, comp).reshape(N_RELS * N_NODES, O)
    acc, deg = _sc_aggregate(y, gidx_r, dst_r)
    return _finalize(acc, deg)[:N_NODES]


# speedup vs baseline: 1.5927x; 1.0536x over previous
import functools

import jax
import jax.numpy as jnp
from jax import lax
from jax.experimental import pallas as pl
from jax.experimental.pallas import tpu as pltpu
from jax.experimental.pallas import tpu_sc as plsc

N_NODES = 10000
H = 128
O = 128
N_RELS = 8
N_BASES = 4
N_EDGES = 320000

NC = 2
NS = 16
NW = NC * NS
E_PER_W = N_EDGES // NW
CHUNK = 40
N_CHUNKS = E_PER_W // CHUNK
ACC_ROWS = 10240
ROWS_PER_S = ACC_ROWS // NS


def _make_y(x, V, comp):
    nb = 10
    bm = N_NODES // nb

    def body(comp_ref, v_ref, x_ref, y_ref):
        xb = x_ref[...]
        for r in range(N_RELS):
            w = jnp.sum(comp_ref[r][:, None, None] * v_ref[...], axis=0)
            y_ref[r] = jnp.dot(xb, w, preferred_element_type=jnp.float32)

    return pl.pallas_call(
        body,
        grid=(nb,),
        in_specs=[
            pl.BlockSpec((N_RELS, N_BASES), lambda n: (0, 0)),
            pl.BlockSpec((N_BASES, H, O), lambda n: (0, 0, 0)),
            pl.BlockSpec((bm, H), lambda n: (n, 0)),
        ],
        out_specs=pl.BlockSpec((N_RELS, bm, O), lambda n: (0, n, 0)),
        out_shape=jax.ShapeDtypeStruct((N_RELS, N_NODES, O), jnp.float32),
    )(comp, V, x)


def _sc_aggregate(y, edge_index, edge_type):
    mesh = plsc.VectorSubcoreMesh(core_axis_name="c", subcore_axis_name="s",
                                  num_cores=NC, num_subcores=NS)

    RING = 5

    @functools.partial(
        pl.kernel,
        mesh=mesh,
        compiler_params=pltpu.CompilerParams(use_tc_tiling_on_sc=False,
                                             needs_layout_passes=False),
        out_type=(
            jax.ShapeDtypeStruct((NC, ACC_ROWS, O), jnp.float32),
            jax.ShapeDtypeStruct((NW, ACC_ROWS), jnp.float32),
        ),
        scratch_types=[
            pltpu.VMEM((E_PER_W,), jnp.int32),
            pltpu.VMEM((2000,), jnp.int32),
            pltpu.VMEM((RING * CHUNK + 8,), jnp.int32),
            pltpu.VMEM((RING, CHUNK, O), jnp.float32),
            pltpu.VMEM((ACC_ROWS,), jnp.float32),
            pltpu.VMEM_SHARED((ACC_ROWS, O), jnp.float32),
            pltpu.SemaphoreType.DMA,
            pltpu.SemaphoreType.DMA,
        ],
    )
    def body(y_hbm, ei_hbm, et_hbm, acc_out, deg_out,
             gidx_b, et_b, dst_b, rows_b, hist, acc_sh, gsem, dsem):
        c = lax.axis_index("c")
        s = lax.axis_index("s")
        wid = c * NS + s
        base = s * ROWS_PER_S

        zeros16 = jnp.zeros((16,), jnp.float32)
        ones16 = jnp.ones((16,), jnp.float32)

        def z_rows(i, _):
            rows_b[0, i // 8, pl.ds((i % 8) * 16, 16)] = zeros16
            return 0
        lax.fori_loop(0, CHUNK * (O // 16), z_rows, 0)

        def z_hist(i, _):
            hist[pl.ds(i * 16, 16)] = zeros16
            return 0
        lax.fori_loop(0, ACC_ROWS // 16, z_hist, 0)

        for k in range(ROWS_PER_S // CHUNK):
            pltpu.sync_copy(rows_b.at[0],
                            acc_sh.at[pl.ds(base + k * CHUNK, CHUNK)])

        ebase = wid * E_PER_W
        pltpu.sync_copy(ei_hbm.at[0, pl.ds(ebase, E_PER_W)], gidx_b)
        for blk in range(E_PER_W // 2000):
            pltpu.sync_copy(et_hbm.at[pl.ds(ebase + blk * 2000, 2000)], et_b)

            def gidx(i, _, blk=blk):
                sl = pl.ds(blk * 2000 + i * 16, 16)
                gidx_b[sl] = et_b[pl.ds(i * 16, 16)] * N_NODES + gidx_b[sl]
                return 0
            lax.fori_loop(0, 125, gidx, 0)

        plsc.subcore_barrier()

        for p in range(RING - 1):
            pltpu.async_copy(ei_hbm.at[1, pl.ds(ebase + p * CHUNK, CHUNK)],
                             dst_b.at[pl.ds(p * CHUNK, CHUNK)], dsem)
            pltpu.async_copy(y_hbm.at[gidx_b.at[pl.ds(p * CHUNK, CHUNK)]],
                             rows_b.at[p], gsem)

        mask8 = lax.iota(jnp.int32, 16) < 8

        def hist_chunk(b):
            for i in range(2):
                d = dst_b[pl.ds(b * CHUNK + i * 16, 16)]
                plsc.addupdate_scatter(hist, [d], ones16)
            d = dst_b[pl.ds(b * CHUNK + 32, 16)]
            plsc.addupdate_scatter(hist, [d], ones16, mask=mask8)

        def do_chunk(j, b):
            pltpu.make_async_copy(ei_hbm.at[1, pl.ds(ebase, CHUNK)],
                                  dst_b.at[pl.ds(b * CHUNK, CHUNK)],
                                  dsem).wait()
            pltpu.make_async_copy(
                y_hbm.at[gidx_b.at[pl.ds(0, CHUNK)]], rows_b.at[b],
                gsem).wait()

            hist_chunk(b)

            @pl.when(j + RING - 1 < N_CHUNKS)
            def _():
                jj = j + RING - 1
                nb_ = (b + RING - 1) % RING
                pltpu.async_copy(
                    ei_hbm.at[1, pl.ds(ebase + jj * CHUNK, CHUNK)],
                    dst_b.at[pl.ds(nb_ * CHUNK, CHUNK)], dsem)
                idx = gidx_b.at[pl.ds(jj * CHUNK, CHUNK)]
                pltpu.async_copy(y_hbm.at[idx], rows_b.at[nb_], gsem)

            pltpu.sync_copy(rows_b.at[b],
                            acc_sh.at[dst_b.at[pl.ds(b * CHUNK, CHUNK)]],
                            add=True)

        def outer(t, _):
            for u in range(RING):
                do_chunk(RING * t + u, u)
            return 0
        lax.fori_loop(0, N_CHUNKS // RING, outer, 0)

        for j in range((N_CHUNKS // RING) * RING, N_CHUNKS):
            do_chunk(j, j % RING)

        plsc.subcore_barrier()

        pltpu.sync_copy(acc_sh.at[pl.ds(base, ROWS_PER_S)],
                        acc_out.at[c, pl.ds(base, ROWS_PER_S)])
        pltpu.sync_copy(hist, deg_out.at[wid])

    return body(y, edge_index, edge_type)


def _finalize(acc, deg):
    nb = 5
    bm = ACC_ROWS // nb

    def body(a_ref, d_ref, o_ref):
        d = jnp.sum(d_ref[...], axis=0)[:, None]
        norm = 1.0 / jnp.maximum(d, 1.0)
        o_ref[...] = jnp.maximum((a_ref[0] + a_ref[1]) * norm, 0.0)

    return pl.pallas_call(
        body,
        grid=(nb,),
        in_specs=[
            pl.BlockSpec((NC, bm, O), lambda n: (0, n, 0)),
            pl.BlockSpec((NW, bm), lambda n: (0, n)),
        ],
        out_specs=pl.BlockSpec((bm, O), lambda n: (n, 0)),
        out_shape=jax.ShapeDtypeStruct((N_NODES, O), jnp.float32),
    )(acc, deg)


def kernel(x, edge_index, edge_type, V, comp):
    y = _make_y(x, V, comp).reshape(N_RELS * N_NODES, O)
    acc, deg = _sc_aggregate(y, edge_index.astype(jnp.int32),
                             edge_type.astype(jnp.int32))
    return _finalize(acc, deg)
